# Initial kernel scaffold; baseline (speedup 1.0000x reference)
import functools

import jax
import jax.numpy as jnp
from jax import lax
from jax.experimental import pallas as pl
from jax.experimental.pallas import tpu as pltpu
from jax.experimental.pallas import tpu_sc as plsc

NEG = -3.0e38


def _normalize_rows(a):
    m = jnp.mean(a, axis=1, keepdims=True)
    c = a - m
    n = jnp.sqrt(jnp.sum(c * c, axis=1, keepdims=True))
    return c / jnp.maximum(n, 1e-12)


def _sims_body(nk, kb, x_ref, k_ref, o_ref):
    j = pl.program_id(1)
    qn = _normalize_rows(x_ref[...])
    rn = _normalize_rows(k_ref[...])
    s = lax.dot_general(qn, rn, (((1,), (1,)), ((), ())),
                        preferred_element_type=jnp.float32)
    lane = lax.broadcasted_iota(jnp.int32, s.shape, 1)
    s = jnp.where(lane < nk - j * kb, s, NEG)
    o_ref[...] = s


def _sims_call(x, keys, nkp, qt, kb, interpret=False):
    nq, d = x.shape
    nk = keys.shape[0]
    grid = (nq // qt, nkp // kb)
    return pl.pallas_call(
        functools.partial(_sims_body, nk, kb),
        grid=grid,
        in_specs=[
            pl.BlockSpec((qt, d), lambda i, j: (i, 0)),
            pl.BlockSpec((kb, d), lambda i, j: (j, 0)),
        ],
        out_specs=pl.BlockSpec((qt, kb), lambda i, j: (i, j)),
        out_shape=jax.ShapeDtypeStruct((nq, nkp), jnp.float32),
        compiler_params=pltpu.CompilerParams(
            dimension_semantics=("parallel", "arbitrary")),
        interpret=interpret,
    )(x, keys)


def _chunksel_body(g, k, s_ref, cid_ref):
    s = s_ref[...]
    rows = s.shape[0]
    c = s.shape[1] // g
    cm = jnp.max(s.reshape(rows, c, g), axis=2)
    ci = lax.broadcasted_iota(jnp.int32, (rows, c), 1)
    ids = []
    for _ in range(k):
        m = jnp.max(cm, axis=1, keepdims=True)
        pos = jnp.min(jnp.where(cm == m, ci, c), axis=1, keepdims=True)
        ids.append(pos)
        cm = jnp.where(ci == pos, NEG, cm)
    cid_ref[...] = jnp.concatenate(ids, axis=1)


def _chunksel_call(sims, g, k, qb, interpret=False):
    nq, nkp = sims.shape
    return pl.pallas_call(
        functools.partial(_chunksel_body, g, k),
        grid=(nq // qb,),
        in_specs=[pl.BlockSpec((qb, nkp), lambda i: (i, 0))],
        out_specs=pl.BlockSpec((qb, k), lambda i: (i, 0)),
        out_shape=jax.ShapeDtypeStruct((nq, k), jnp.int32),
        interpret=interpret,
    )(sims)


def _final_body(g, k, cand_ref, cid_ref, w_ref, idx_ref):
    v = cand_ref[...]
    cid = cid_ref[...]
    rows, nc = v.shape
    pi = lax.broadcasted_iota(jnp.int32, (rows, nc), 1)
    rk = lax.broadcasted_iota(jnp.int32, (rows, k), 1)
    vals, gidx = [], []
    for _ in range(k):
        m = jnp.max(v, axis=1, keepdims=True)
        pos = jnp.min(jnp.where(v == m, pi, nc), axis=1, keepdims=True)
        vals.append(m)
        rr = pos // g
        off = pos % g
        sel = jnp.sum(jnp.where(rk == rr, cid, 0), axis=1, keepdims=True)
        gidx.append(sel * g + off)
        v = jnp.where(pi == pos, NEG, v)
    vals = jnp.concatenate(vals, axis=1)
    e = jnp.exp(vals - vals[:, :1])
    w_ref[...] = e / jnp.sum(e, axis=1, keepdims=True)
    idx_ref[...] = jnp.concatenate(gidx, axis=1)


def _final_call(cand, cid, g, k, qb, interpret=False):
    nq = cand.shape[0]
    return pl.pallas_call(
        functools.partial(_final_body, g, k),
        grid=(nq // qb,),
        in_specs=[
            pl.BlockSpec((qb, cand.shape[1]), lambda i: (i, 0)),
            pl.BlockSpec((qb, k), lambda i: (i, 0)),
        ],
        out_specs=[
            pl.BlockSpec((qb, k), lambda i: (i, 0)),
            pl.BlockSpec((qb, k), lambda i: (i, 0)),
        ],
        out_shape=[
            jax.ShapeDtypeStruct((nq, k), jnp.float32),
            jax.ShapeDtypeStruct((nq, k), jnp.int32),
        ],
        interpret=interpret,
    )(cand, cid)


def _sc_gather(table, idx, chunk):
    b = idx.shape[0]
    d = table.shape[1]
    info = plsc.get_sparse_core_info()
    ncores, nsub = info.num_cores, info.num_subcores
    nw = ncores * nsub
    bpw = b // nw
    steps = bpw // chunk
    mesh = plsc.VectorSubcoreMesh(core_axis_name="c", subcore_axis_name="s")

    @functools.partial(
        pl.kernel, mesh=mesh,
        out_type=jax.ShapeDtypeStruct((b, d), jnp.float32),
        scratch_types=[
            pltpu.VMEM((chunk,), jnp.int32),
            pltpu.VMEM((chunk, d), jnp.float32),
            pltpu.SemaphoreType.DMA,
        ],
    )
    def k(table_hbm, idx_hbm, out_hbm, idx_v, rows_v, sem):
        wid = lax.axis_index("s") * ncores + lax.axis_index("c")

        def body(i, carry):
            base = wid * bpw + i * chunk
            pltpu.sync_copy(idx_hbm.at[pl.ds(base, chunk)], idx_v)
            pltpu.async_copy(table_hbm.at[idx_v], rows_v, sem).wait()
            pltpu.sync_copy(rows_v, out_hbm.at[pl.ds(base, chunk)])
            return carry

        lax.fori_loop(0, steps, body, 0)

    return k(table, idx)


def _impl(x, X_train, Y_train, *, qt, kb, g, qb_sel, qb_fin, sc_chunk,
          interpret=False, sc=True):
    nq, d = x.shape
    nk = Y_train.shape[0]
    k = 20
    nkp = ((nk + kb - 1) // kb) * kb
    c = nkp // g

    sims = _sims_call(x, X_train, nkp, qt, kb, interpret)
    cid = _chunksel_call(sims, g, k, qb_sel, interpret)

    flat_cid = (jnp.arange(nq, dtype=jnp.int32)[:, None] * c + cid).reshape(-1)
    table = sims.reshape(nq * c, g)
    if sc:
        cand = _sc_gather(table, flat_cid, sc_chunk)
    else:
        cand = jnp.take(table, flat_cid, axis=0)
    cand = cand.reshape(nq, k * g)

    w, idx = _final_call(cand, cid, g, k, qb_fin, interpret)
    flat_idx = idx.reshape(-1)
    if sc:
        yk = _sc_gather(Y_train, flat_idx, sc_chunk)
    else:
        yk = jnp.take(Y_train, flat_idx, axis=0)
    return w, yk.reshape(nq, k, Y_train.shape[1])


def kernel(x, X_train, Y_train):
    return _impl(x, X_train, Y_train, qt=2048, kb=2048, g=64,
                 qb_sel=64, qb_fin=256, sc_chunk=1280)


# speedup vs baseline: 2.4374x; 2.4374x over previous
import functools

import jax
import jax.numpy as jnp
from jax import lax
from jax.experimental import pallas as pl
from jax.experimental.pallas import tpu as pltpu
from jax.experimental.pallas import tpu_sc as plsc

NEG = -3.0e38
G = 128


def _sims_body(nk, kb, x_ref, k_ref, o_ref, cm_ref):
    j = pl.program_id(1)
    s = lax.dot_general(x_ref[...], k_ref[...], (((1,), (1,)), ((), ())),
                        preferred_element_type=jnp.float32)
    lane = lax.broadcasted_iota(jnp.int32, s.shape, 1)
    s = jnp.where(lane < nk - j * kb, s, NEG)
    o_ref[...] = s
    qt = s.shape[0]
    cm = jnp.max(s.reshape(qt, kb // G, G), axis=2)
    cm_ref[...] = cm.reshape(1, qt, kb // G)


def _sims_call(x, keys, nkp, qt, kb, interpret=False):
    nq, d = x.shape
    nk = keys.shape[0]
    nsec = nkp // kb
    cpb = kb // G
    return pl.pallas_call(
        functools.partial(_sims_body, nk, kb),
        grid=(nq // qt, nsec),
        in_specs=[
            pl.BlockSpec((qt, d), lambda i, j: (i, 0)),
            pl.BlockSpec((kb, d), lambda i, j: (j, 0)),
        ],
        out_specs=[
            pl.BlockSpec((qt, kb), lambda i, j: (i, j)),
            pl.BlockSpec((1, qt, cpb), lambda i, j: (j, i, 0)),
        ],
        out_shape=[
            jax.ShapeDtypeStruct((nq, nkp), jnp.float32),
            jax.ShapeDtypeStruct((nsec, nq, cpb), jnp.float32),
        ],
        compiler_params=pltpu.CompilerParams(
            dimension_semantics=("parallel", "arbitrary")),
        interpret=interpret,
    )(x, keys)


def _chunksel_body(k, cm_ref, cid_ref):
    cm = cm_ref[...]
    nsec, qb, cpb = cm.shape
    sec_i = lax.broadcasted_iota(jnp.int32, (nsec, qb, cpb), 0)
    sub_i = lax.broadcasted_iota(jnp.int32, (nsec, qb, cpb), 2)
    sec2_i = lax.broadcasted_iota(jnp.int32, (nsec, qb), 0)
    ids = []
    for _ in range(k):
        m1 = jnp.max(cm, axis=2)
        m = jnp.max(m1, axis=0, keepdims=True)
        sec = jnp.min(jnp.where(m1 == m, sec2_i, nsec),
                      axis=0, keepdims=True)
        msk = (cm == m[:, :, None]) & (sec_i == sec[:, :, None])
        t = jnp.min(jnp.where(msk, sub_i, cpb), axis=2)
        sub = jnp.min(t, axis=0, keepdims=True)
        ids.append(sec * cpb + sub)
        kill = (sec_i == sec[:, :, None]) & (sub_i == sub[:, :, None])
        cm = jnp.where(kill, NEG, cm)
    cid_ref[...] = jnp.concatenate(ids, axis=0)


def _chunksel_call(cm3, k, qb, interpret=False):
    nsec, nq, cpb = cm3.shape
    return pl.pallas_call(
        functools.partial(_chunksel_body, k),
        grid=(nq // qb,),
        in_specs=[pl.BlockSpec((nsec, qb, cpb), lambda i: (0, i, 0))],
        out_specs=pl.BlockSpec((k, qb), lambda i: (0, i)),
        out_shape=jax.ShapeDtypeStruct((k, nq), jnp.int32),
        interpret=interpret,
    )(cm3)


def _final_body(k, cand_ref, cid_ref, w_ref, idx_ref):
    v = cand_ref[...]
    cid = cid_ref[...]
    rows, nc = v.shape
    pi = lax.broadcasted_iota(jnp.int32, (rows, nc), 1)
    rk = lax.broadcasted_iota(jnp.int32, (rows, k), 1)
    vals, gidx = [], []
    for _ in range(k):
        m = jnp.max(v, axis=1, keepdims=True)
        pos = jnp.min(jnp.where(v == m, pi, nc), axis=1, keepdims=True)
        vals.append(m)
        rr = pos // G
        off = pos % G
        sel = jnp.sum(jnp.where(rk == rr, cid, 0), axis=1, keepdims=True)
        gidx.append(sel * G + off)
        v = jnp.where(pi == pos, NEG, v)
    vals = jnp.concatenate(vals, axis=1)
    e = jnp.exp(vals - vals[:, :1])
    w_ref[...] = e / jnp.sum(e, axis=1, keepdims=True)
    idx_ref[...] = jnp.concatenate(gidx, axis=1)


def _final_call(cand, cid, k, qb, interpret=False):
    nq = cand.shape[0]
    return pl.pallas_call(
        functools.partial(_final_body, k),
        grid=(nq // qb,),
        in_specs=[
            pl.BlockSpec((qb, cand.shape[1]), lambda i: (i, 0)),
            pl.BlockSpec((qb, k), lambda i: (i, 0)),
        ],
        out_specs=[
            pl.BlockSpec((qb, k), lambda i: (i, 0)),
            pl.BlockSpec((qb, k), lambda i: (i, 0)),
        ],
        out_shape=[
            jax.ShapeDtypeStruct((nq, k), jnp.float32),
            jax.ShapeDtypeStruct((nq, k), jnp.int32),
        ],
        interpret=interpret,
    )(cand, cid)


def _sc_gather(table, idx, chunk):
    b = idx.shape[0]
    d = table.shape[1]
    info = plsc.get_sparse_core_info()
    ncores, nsub = info.num_cores, info.num_subcores
    nw = ncores * nsub
    bpw = b // nw
    steps = bpw // chunk
    mesh = plsc.VectorSubcoreMesh(core_axis_name="c", subcore_axis_name="s")

    @functools.partial(
        pl.kernel, mesh=mesh,
        out_type=jax.ShapeDtypeStruct((b, d), jnp.float32),
        scratch_types=[
            pltpu.VMEM((chunk,), jnp.int32),
            pltpu.VMEM((chunk, d), jnp.float32),
            pltpu.SemaphoreType.DMA,
        ],
    )
    def k(table_hbm, idx_hbm, out_hbm, idx_v, rows_v, sem):
        wid = lax.axis_index("s") * ncores + lax.axis_index("c")

        def body(i, carry):
            base = wid * bpw + i * chunk
            pltpu.sync_copy(idx_hbm.at[pl.ds(base, chunk)], idx_v)
            pltpu.async_copy(table_hbm.at[idx_v], rows_v, sem).wait()
            pltpu.sync_copy(rows_v, out_hbm.at[pl.ds(base, chunk)])
            return carry

        lax.fori_loop(0, steps, body, 0)

    return k(table, idx)


def _row_encode_normalize(a):
    mean = a.mean(axis=-1, keepdims=True)
    std = jnp.std(a, axis=-1, keepdims=True, ddof=1) + 1e-06
    xn = (a - mean) / std
    n = jnp.linalg.norm(xn, axis=1, keepdims=True)
    return xn / jnp.maximum(n, 1e-12)


def _impl(x, X_train, Y_train, *, qt, kb, qb_sel, qb_fin, sc_chunk,
          interpret=False, sc=True):
    nq, d = x.shape
    nk = Y_train.shape[0]
    k = 20
    nkp = ((nk + kb - 1) // kb) * kb
    c = nkp // G

    qn = _row_encode_normalize(x)
    rn = _row_encode_normalize(X_train)
    sims, cm3 = _sims_call(qn, rn, nkp, qt, kb, interpret)
    cid_t = _chunksel_call(cm3, k, qb_sel, interpret)
    cid = cid_t.T

    flat_cid = (jnp.arange(nq, dtype=jnp.int32)[:, None] * c + cid).reshape(-1)
    table = sims.reshape(nq * c, G)
    if sc:
        cand = _sc_gather(table, flat_cid, sc_chunk)
    else:
        cand = jnp.take(table, flat_cid, axis=0)
    cand = cand.reshape(nq, k * G)

    w, idx = _final_call(cand, cid, k, qb_fin, interpret)
    flat_idx = idx.reshape(-1)
    dy = Y_train.shape[1]
    if sc:
        ypad = jnp.pad(Y_train, ((0, 0), (0, 128 - dy))) if dy < 128 else Y_train
        yk = _sc_gather(ypad, flat_idx, sc_chunk)[:, :dy]
    else:
        yk = jnp.take(Y_train, flat_idx, axis=0)
    return w, yk.reshape(nq, k, dy)


def kernel(x, X_train, Y_train):
    return _impl(x, X_train, Y_train, qt=2048, kb=1024,
                 qb_sel=256, qb_fin=256, sc_chunk=640)


# speedup vs baseline: 6.2463x; 2.5627x over previous
import functools

import jax
import jax.numpy as jnp
from jax import lax
from jax.experimental import pallas as pl
from jax.experimental.pallas import tpu as pltpu
from jax.experimental.pallas import tpu_sc as plsc

NEG = -3.0e38
G = 128


def _sims_body(nk, kb, x_ref, k_ref, o_ref, cm_ref):
    j = pl.program_id(1)
    s = lax.dot_general(x_ref[...], k_ref[...], (((1,), (1,)), ((), ())),
                        preferred_element_type=jnp.float32)
    lane = lax.broadcasted_iota(jnp.int32, s.shape, 1)
    s = jnp.where(lane < nk - j * kb, s, NEG)
    o_ref[...] = s
    qt = s.shape[0]
    cm = jnp.max(s.reshape(qt, kb // G, G), axis=2)
    cm_ref[...] = cm.reshape(1, qt, kb // G)


def _sims_call(x, keys, nkp, qt, kb, interpret=False):
    nq, d = x.shape
    nk = keys.shape[0]
    nsec = nkp // kb
    cpb = kb // G
    return pl.pallas_call(
        functools.partial(_sims_body, nk, kb),
        grid=(nq // qt, nsec),
        in_specs=[
            pl.BlockSpec((qt, d), lambda i, j: (i, 0)),
            pl.BlockSpec((kb, d), lambda i, j: (j, 0)),
        ],
        out_specs=[
            pl.BlockSpec((qt, kb), lambda i, j: (i, j)),
            pl.BlockSpec((1, qt, cpb), lambda i, j: (j, i, 0)),
        ],
        out_shape=[
            jax.ShapeDtypeStruct((nq, nkp), jnp.float32),
            jax.ShapeDtypeStruct((nsec, nq, cpb), jnp.float32),
        ],
        compiler_params=pltpu.CompilerParams(
            dimension_semantics=("parallel", "arbitrary")),
        interpret=interpret,
    )(x, keys)


def _chunksel_body(k, cm_ref, cid_ref):
    cm = cm_ref[...]
    qb, c = cm.shape
    ci = lax.broadcasted_iota(jnp.int32, (qb, c), 1)
    ids = []
    for _ in range(k):
        m = jnp.max(cm, axis=1, keepdims=True)
        pos = jnp.min(jnp.where(cm == m, ci, c), axis=1, keepdims=True)
        ids.append(pos)
        cm = jnp.where(ci == pos, NEG, cm)
    cid_ref[...] = jnp.concatenate(ids, axis=1)


def _chunksel_call(cm2, k, qb, interpret=False):
    nq, c = cm2.shape
    return pl.pallas_call(
        functools.partial(_chunksel_body, k),
        grid=(nq // qb,),
        in_specs=[pl.BlockSpec((qb, c), lambda i: (i, 0))],
        out_specs=pl.BlockSpec((qb, k), lambda i: (i, 0)),
        out_shape=jax.ShapeDtypeStruct((nq, k), jnp.int32),
        interpret=interpret,
    )(cm2)


def _final_body(k, cand_ref, cid_ref, w_ref, idx_ref):
    v = cand_ref[...]
    cid = cid_ref[...]
    rows, nc = v.shape
    pi = lax.broadcasted_iota(jnp.int32, (rows, nc), 1)
    rk = lax.broadcasted_iota(jnp.int32, (rows, k), 1)
    vals, gidx = [], []
    for _ in range(k):
        m = jnp.max(v, axis=1, keepdims=True)
        pos = jnp.min(jnp.where(v == m, pi, nc), axis=1, keepdims=True)
        vals.append(m)
        rr = pos // G
        off = pos % G
        sel = jnp.sum(jnp.where(rk == rr, cid, 0), axis=1, keepdims=True)
        gidx.append(sel * G + off)
        v = jnp.where(pi == pos, NEG, v)
    vals = jnp.concatenate(vals, axis=1)
    e = jnp.exp(vals - vals[:, :1])
    w_ref[...] = e / jnp.sum(e, axis=1, keepdims=True)
    idx_ref[...] = jnp.concatenate(gidx, axis=1)


def _final_call(cand, cid, k, qb, interpret=False):
    nq = cand.shape[0]
    return pl.pallas_call(
        functools.partial(_final_body, k),
        grid=(nq // qb,),
        in_specs=[
            pl.BlockSpec((qb, cand.shape[1]), lambda i: (i, 0)),
            pl.BlockSpec((qb, k), lambda i: (i, 0)),
        ],
        out_specs=[
            pl.BlockSpec((qb, k), lambda i: (i, 0)),
            pl.BlockSpec((qb, k), lambda i: (i, 0)),
        ],
        out_shape=[
            jax.ShapeDtypeStruct((nq, k), jnp.float32),
            jax.ShapeDtypeStruct((nq, k), jnp.int32),
        ],
        interpret=interpret,
    )(cand, cid)


def _sc_gather(table, idx, chunk):
    b = idx.shape[0]
    d = table.shape[1]
    info = plsc.get_sparse_core_info()
    ncores, nsub = info.num_cores, info.num_subcores
    nw = ncores * nsub
    bpw = b // nw
    steps = bpw // chunk
    mesh = plsc.VectorSubcoreMesh(core_axis_name="c", subcore_axis_name="s")

    @functools.partial(
        pl.kernel, mesh=mesh,
        out_type=jax.ShapeDtypeStruct((b, d), jnp.float32),
        scratch_types=[
            pltpu.VMEM((chunk,), jnp.int32),
            pltpu.VMEM((chunk, d), jnp.float32),
            pltpu.SemaphoreType.DMA,
        ],
    )
    def k(table_hbm, idx_hbm, out_hbm, idx_v, rows_v, sem):
        wid = lax.axis_index("s") * ncores + lax.axis_index("c")

        def body(i, carry):
            base = wid * bpw + i * chunk
            pltpu.sync_copy(idx_hbm.at[pl.ds(base, chunk)], idx_v)
            pltpu.async_copy(table_hbm.at[idx_v], rows_v, sem).wait()
            pltpu.sync_copy(rows_v, out_hbm.at[pl.ds(base, chunk)])
            return carry

        lax.fori_loop(0, steps, body, 0)

    return k(table, idx)


def _row_encode_normalize(a):
    mean = a.mean(axis=-1, keepdims=True)
    std = jnp.std(a, axis=-1, keepdims=True, ddof=1) + 1e-06
    xn = (a - mean) / std
    n = jnp.linalg.norm(xn, axis=1, keepdims=True)
    return xn / jnp.maximum(n, 1e-12)


def _impl(x, X_train, Y_train, *, qt, kb, qb_sel, qb_fin, sc_chunk,
          interpret=False, sc=True):
    nq, d = x.shape
    nk = Y_train.shape[0]
    k = 20
    nkp = ((nk + kb - 1) // kb) * kb
    c = nkp // G

    qn = _row_encode_normalize(x)
    rn = _row_encode_normalize(X_train)
    sims, cm3 = _sims_call(qn, rn, nkp, qt, kb, interpret)
    cm2 = cm3.transpose(1, 0, 2).reshape(nq, c)
    cid = _chunksel_call(cm2, k, qb_sel, interpret)

    flat_cid = (jnp.arange(nq, dtype=jnp.int32)[:, None] * c + cid).reshape(-1)
    table = sims.reshape(nq * c, G)
    if sc:
        cand = _sc_gather(table, flat_cid, sc_chunk)
    else:
        cand = jnp.take(table, flat_cid, axis=0)
    cand = cand.reshape(nq, k * G)

    w, idx = _final_call(cand, cid, k, qb_fin, interpret)
    flat_idx = idx.reshape(-1)
    dy = Y_train.shape[1]
    if sc:
        ypad = jnp.pad(Y_train, ((0, 0), (0, 128 - dy))) if dy < 128 else Y_train
        yk = _sc_gather(ypad, flat_idx, sc_chunk)[:, :dy]
    else:
        yk = jnp.take(Y_train, flat_idx, axis=0)
    return w, yk.reshape(nq, k, dy)


def kernel(x, X_train, Y_train):
    return _impl(x, X_train, Y_train, qt=2048, kb=1024,
                 qb_sel=512, qb_fin=256, sc_chunk=640)
